# baseline (speedup 1.0000x reference)
import functools

import jax
import jax.numpy as jnp
from jax import lax
from jax.experimental import pallas as pl
from jax.experimental.pallas import tpu as pltpu
from jax.experimental.pallas import tpu_sc as plsc

EMB_DIM = 32
CATE_NUM = 26
PROJ_DIM = 128
MAX_SEQ_LEN = 50
BATCH = 1024

N_ROWS = BATCH * MAX_SEQ_LEN * CATE_NUM
IN_DIM = EMB_DIM * CATE_NUM
BL = BATCH * MAX_SEQ_LEN
HALF = BL // 2
NSLAB = 13
QUADS = 4

NUM_CORES = 2
NUM_SUBCORES = 16
NUM_TILES = NUM_CORES * NUM_SUBCORES
P_PER_TILE = HALF // NUM_TILES
CHUNK = 128
GROUP = 5
CHUNKS_PER_SLAB = P_PER_TILE * QUADS // CHUNK
CHUNKS_PER_TILE = NSLAB * CHUNKS_PER_SLAB
GROUPS_PER_SLAB = CHUNKS_PER_SLAB // GROUP
GROUPS_PER_TILE = NSLAB * GROUPS_PER_SLAB
GROUP_ROWS = GROUP * CHUNK
SLAB_ROWS = HALF * QUADS


IDX_ROWS = BATCH * MAX_SEQ_LEN * CATE_NUM // 128
TILE_WORDS = P_PER_TILE * CATE_NUM
STAGE = 176


def _sc_gather(idx2d, table):
    mesh = plsc.VectorSubcoreMesh(core_axis_name="c", subcore_axis_name="s")

    @functools.partial(
        pl.kernel,
        mesh=mesh,
        compiler_params=pltpu.CompilerParams(use_tc_tiling_on_sc=False,
                                             needs_layout_passes=False),
        out_type=jax.ShapeDtypeStruct((NSLAB, SLAB_ROWS, EMB_DIM),
                                      jnp.float32),
        scratch_types=[
            pltpu.VMEM((2, STAGE, 128), jnp.int32),
            pltpu.VMEM((2, GROUP, CHUNK), jnp.int32),
            pltpu.VMEM((3, GROUP_ROWS, EMB_DIM), jnp.float32),
            pltpu.SemaphoreType.DMA,
            pltpu.SemaphoreType.DMA,
        ],
    )
    def k(table_hbm, idx_hbm, out_hbm, idx_stage, idx_buf, rows_v, gsem, ssem):
        wid = lax.axis_index("s") * NUM_CORES + lax.axis_index("c")
        base_top = wid * TILE_WORDS
        base_bot = HALF * CATE_NUM + wid * TILE_WORDS
        r_top = jnp.minimum((base_top >> 7) & ~7, IDX_ROWS - STAGE)
        r_bot = jnp.minimum((base_bot >> 7) & ~7, IDX_ROWS - STAGE)
        pltpu.sync_copy(idx_hbm.at[pl.ds(r_top, STAGE)], idx_stage.at[0])
        pltpu.sync_copy(idx_hbm.at[pl.ds(r_bot, STAGE)], idx_stage.at[1])
        off_top = base_top - r_top * 128
        off_bot = base_bot - r_bot * 128
        row_base = wid * P_PER_TILE * QUADS
        iota = lax.iota(jnp.int32, 16)
        qv = lax.bitwise_and(iota, 3)
        dv = lax.shift_right_logical(iota, 2)

        def transform(g, pb):
            j = g // GROUPS_PER_SLAB
            s5 = g - j * GROUPS_PER_SLAB
            c2 = 4 * j + qv
            is_bot = c2 >= CATE_NUM
            half = jnp.where(is_bot, 1, 0)
            cadj = jnp.where(is_bot, c2 - CATE_NUM + off_bot, c2 + off_top)
            for i in range(GROUP):
                dp0 = (s5 * GROUP + i) * 32
                for o in range(CHUNK // 16):
                    word = (dp0 + o * 4 + dv) * CATE_NUM + cadj
                    vals = plsc.load_gather(
                        idx_stage,
                        [half, lax.shift_right_logical(word, 7),
                         lax.bitwise_and(word, 127)])
                    idx_buf[pb, i, pl.ds(o * 16, 16)] = vals

        transform(0, 0)

        def body(g, carry):
            j = g // GROUPS_PER_SLAB
            s5 = g - j * GROUPS_PER_SLAB
            p3 = lax.rem(g, 3)
            pi = lax.rem(g, 2)
            buf = rows_v.at[p3]
            @pl.when(g >= 3)
            def _():
                pltpu.make_async_copy(
                    rows_v.at[0],
                    out_hbm.at[0, pl.ds(0, GROUP_ROWS)],
                    ssem).wait()
            cps = []
            for i in range(GROUP):
                cps.append(pltpu.async_copy(
                    table_hbm.at[idx_buf.at[pi, i]],
                    buf.at[pl.ds(i * CHUNK, CHUNK)],
                    gsem))
            transform(g + 1, 1 - pi)
            for cp in cps:
                cp.wait()
            pltpu.async_copy(
                buf,
                out_hbm.at[j, pl.ds(row_base + s5 * GROUP_ROWS, GROUP_ROWS)],
                ssem)
            return carry

        lax.fori_loop(0, GROUPS_PER_TILE, body, 0)
        for _ in range(3):
            pltpu.make_async_copy(
                rows_v.at[0],
                out_hbm.at[0, pl.ds(0, GROUP_ROWS)],
                ssem).wait()

    return k(table, idx2d)


BB = 16
PAIRS = BB * MAX_SEQ_LEN
XW = 7 * 128


def _tc_proj_body(x_ref, wt_ref, wb_ref, b_ref, g_ref, be_ref, o_ref):
    x3 = x_ref[...]
    x_top = jnp.concatenate([x3[j] for j in range(7)], axis=1)
    x_bot = jnp.concatenate([x3[j] for j in range(6, 13)], axis=1)
    bvec = b_ref[...]
    gvec = g_ref[...]
    bevec = be_ref[...]

    def norm(h):
        h = h + bvec
        mu = jnp.mean(h, axis=1, keepdims=True)
        d = h - mu
        var = jnp.mean(d * d, axis=1, keepdims=True)
        out = d * lax.rsqrt(var + 1e-5) * gvec + bevec
        return out.reshape(BB, MAX_SEQ_LEN, PROJ_DIM)

    h_top = jnp.dot(x_top, wt_ref[...], preferred_element_type=jnp.float32)
    h_bot = jnp.dot(x_bot, wb_ref[...], preferred_element_type=jnp.float32)
    o_ref[0] = norm(h_top)
    o_ref[1] = norm(h_bot)


def _tc_proj(x3, W, b, gamma, beta):
    w_top = jnp.concatenate([W, jnp.zeros((64, PROJ_DIM), W.dtype)], axis=0)
    w_bot = jnp.concatenate([jnp.zeros((64, PROJ_DIM), W.dtype), W], axis=0)
    grid = (BATCH // 2 // BB,)
    return pl.pallas_call(
        _tc_proj_body,
        grid=grid,
        in_specs=[
            pl.BlockSpec((NSLAB, PAIRS, 128), lambda i: (0, i, 0)),
            pl.BlockSpec((XW, PROJ_DIM), lambda i: (0, 0)),
            pl.BlockSpec((XW, PROJ_DIM), lambda i: (0, 0)),
            pl.BlockSpec((1, PROJ_DIM), lambda i: (0, 0)),
            pl.BlockSpec((1, PROJ_DIM), lambda i: (0, 0)),
            pl.BlockSpec((1, PROJ_DIM), lambda i: (0, 0)),
        ],
        out_specs=pl.BlockSpec((2, BB, MAX_SEQ_LEN, PROJ_DIM),
                               lambda i: (0, i, 0, 0)),
        out_shape=jax.ShapeDtypeStruct(
            (2, BATCH // 2, MAX_SEQ_LEN, PROJ_DIM), jnp.float32),
    )(x3, w_top, w_bot, b.reshape(1, PROJ_DIM), gamma.reshape(1, PROJ_DIM),
      beta.reshape(1, PROJ_DIM))


def kernel(cate_x, emb_table, W, b, gamma, beta):
    cate_x, emb_table = lax.optimization_barrier((cate_x, emb_table))
    idx2d = cate_x.reshape(IDX_ROWS, 128)
    slabs = _sc_gather(idx2d, emb_table)
    x3 = slabs.reshape(NSLAB, HALF, 128)
    out = _tc_proj(x3, W, b, gamma, beta)
    return out.reshape(BATCH, MAX_SEQ_LEN, PROJ_DIM)


# speedup vs baseline: 1.1010x; 1.1010x over previous
import functools

import jax
import jax.numpy as jnp
from jax import lax
from jax.experimental import pallas as pl
from jax.experimental.pallas import tpu as pltpu
from jax.experimental.pallas import tpu_sc as plsc

EMB_DIM = 32
CATE_NUM = 26
PROJ_DIM = 128
MAX_SEQ_LEN = 50
BATCH = 1024

N_ROWS = BATCH * MAX_SEQ_LEN * CATE_NUM
IN_DIM = EMB_DIM * CATE_NUM
BL = BATCH * MAX_SEQ_LEN
HALF = BL // 2
NSLAB = 13
QUADS = 4

NUM_CORES = 2
NUM_SUBCORES = 16
NUM_TILES = NUM_CORES * NUM_SUBCORES
P_PER_TILE = HALF // NUM_TILES
CHUNK = 128
GROUP = 5
CHUNKS_PER_SLAB = P_PER_TILE * QUADS // CHUNK
CHUNKS_PER_TILE = NSLAB * CHUNKS_PER_SLAB
GROUPS_PER_SLAB = CHUNKS_PER_SLAB // GROUP
GROUPS_PER_TILE = NSLAB * GROUPS_PER_SLAB
GROUP_ROWS = GROUP * CHUNK
SLAB_ROWS = HALF * QUADS


IDX_ROWS = BATCH * MAX_SEQ_LEN * CATE_NUM // 128
TILE_WORDS = P_PER_TILE * CATE_NUM
STAGE = 176


def _sc_gather(idx2d, table):
    mesh = plsc.VectorSubcoreMesh(core_axis_name="c", subcore_axis_name="s")

    @functools.partial(
        pl.kernel,
        mesh=mesh,
        compiler_params=pltpu.CompilerParams(use_tc_tiling_on_sc=False,
                                             needs_layout_passes=False),
        out_type=jax.ShapeDtypeStruct((NSLAB, SLAB_ROWS, EMB_DIM),
                                      jnp.float32),
        scratch_types=[
            pltpu.VMEM((2, STAGE, 128), jnp.int32),
            pltpu.VMEM((2, GROUP, CHUNK), jnp.int32),
            pltpu.VMEM((3, GROUP_ROWS, EMB_DIM), jnp.float32),
            pltpu.SemaphoreType.DMA,
            pltpu.SemaphoreType.DMA,
        ],
    )
    def k(table_hbm, idx_hbm, out_hbm, idx_stage, idx_buf, rows_v, gsem, ssem):
        wid = lax.axis_index("s") * NUM_CORES + lax.axis_index("c")
        base_top = wid * TILE_WORDS
        base_bot = HALF * CATE_NUM + wid * TILE_WORDS
        r_top = jnp.minimum((base_top >> 7) & ~7, IDX_ROWS - STAGE)
        r_bot = jnp.minimum((base_bot >> 7) & ~7, IDX_ROWS - STAGE)
        pltpu.sync_copy(idx_hbm.at[pl.ds(r_top, STAGE)], idx_stage.at[0])
        pltpu.sync_copy(idx_hbm.at[pl.ds(r_bot, STAGE)], idx_stage.at[1])
        off_top = base_top - r_top * 128
        off_bot = base_bot - r_bot * 128
        row_base = wid * P_PER_TILE * QUADS
        iota = lax.iota(jnp.int32, 16)
        qv = lax.bitwise_and(iota, 3)
        dv = lax.shift_right_logical(iota, 2)

        def transform(g, pb):
            j = g // GROUPS_PER_SLAB
            s5 = g - j * GROUPS_PER_SLAB
            c2 = 4 * j + qv
            is_bot = c2 >= CATE_NUM
            half = jnp.where(is_bot, 1, 0)
            cadj = jnp.where(is_bot, c2 - CATE_NUM + off_bot, c2 + off_top)
            for i in range(GROUP):
                dp0 = (s5 * GROUP + i) * 32
                for o in range(CHUNK // 16):
                    word = (dp0 + o * 4 + dv) * CATE_NUM + cadj
                    vals = plsc.load_gather(
                        idx_stage,
                        [half, lax.shift_right_logical(word, 7),
                         lax.bitwise_and(word, 127)])
                    idx_buf[pb, i, pl.ds(o * 16, 16)] = vals

        transform(0, 0)

        def body(g, carry):
            j = g // GROUPS_PER_SLAB
            s5 = g - j * GROUPS_PER_SLAB
            p3 = lax.rem(g, 3)
            pi = lax.rem(g, 2)
            buf = rows_v.at[p3]
            @pl.when(g >= 3)
            def _():
                pltpu.make_async_copy(
                    rows_v.at[0],
                    out_hbm.at[0, pl.ds(0, GROUP_ROWS)],
                    ssem).wait()
            cps = []
            for i in range(GROUP):
                cps.append(pltpu.async_copy(
                    table_hbm.at[idx_buf.at[pi, i]],
                    buf.at[pl.ds(i * CHUNK, CHUNK)],
                    gsem))
            transform(g + 1, 1 - pi)
            for cp in cps:
                cp.wait()
            pltpu.async_copy(
                buf,
                out_hbm.at[j, pl.ds(row_base + s5 * GROUP_ROWS, GROUP_ROWS)],
                ssem)
            return carry

        lax.fori_loop(0, GROUPS_PER_TILE, body, 0)
        for _ in range(3):
            pltpu.make_async_copy(
                rows_v.at[0],
                out_hbm.at[0, pl.ds(0, GROUP_ROWS)],
                ssem).wait()

    return k(table, idx2d)


BB = 32
PAIRS = BB * MAX_SEQ_LEN
XW = 7 * 128


def _tc_proj_body(x_ref, wt_ref, wb_ref, b_ref, g_ref, be_ref, o_ref):
    x3 = x_ref[...]
    x_top = jnp.concatenate([x3[j] for j in range(7)], axis=1)
    x_bot = jnp.concatenate([x3[j] for j in range(6, 13)], axis=1)
    bvec = b_ref[...]
    gvec = g_ref[...]
    bevec = be_ref[...]

    def norm(h):
        h = h + bvec
        mu = jnp.mean(h, axis=1, keepdims=True)
        d = h - mu
        var = jnp.mean(d * d, axis=1, keepdims=True)
        out = d * lax.rsqrt(var + 1e-5) * gvec + bevec
        return out.reshape(BB, MAX_SEQ_LEN, PROJ_DIM)

    h_top = jnp.dot(x_top, wt_ref[...], preferred_element_type=jnp.float32)
    h_bot = jnp.dot(x_bot, wb_ref[...], preferred_element_type=jnp.float32)
    o_ref[0] = norm(h_top)
    o_ref[1] = norm(h_bot)


def _tc_proj(x3, W, b, gamma, beta):
    w_top = jnp.concatenate([W, jnp.zeros((64, PROJ_DIM), W.dtype)], axis=0)
    w_bot = jnp.concatenate([jnp.zeros((64, PROJ_DIM), W.dtype), W], axis=0)
    grid = (BATCH // 2 // BB,)
    return pl.pallas_call(
        _tc_proj_body,
        grid=grid,
        in_specs=[
            pl.BlockSpec((NSLAB, PAIRS, 128), lambda i: (0, i, 0)),
            pl.BlockSpec((XW, PROJ_DIM), lambda i: (0, 0)),
            pl.BlockSpec((XW, PROJ_DIM), lambda i: (0, 0)),
            pl.BlockSpec((1, PROJ_DIM), lambda i: (0, 0)),
            pl.BlockSpec((1, PROJ_DIM), lambda i: (0, 0)),
            pl.BlockSpec((1, PROJ_DIM), lambda i: (0, 0)),
        ],
        out_specs=pl.BlockSpec((2, BB, MAX_SEQ_LEN, PROJ_DIM),
                               lambda i: (0, i, 0, 0)),
        out_shape=jax.ShapeDtypeStruct(
            (2, BATCH // 2, MAX_SEQ_LEN, PROJ_DIM), jnp.float32),
    )(x3, w_top, w_bot, b.reshape(1, PROJ_DIM), gamma.reshape(1, PROJ_DIM),
      beta.reshape(1, PROJ_DIM))


def kernel(cate_x, emb_table, W, b, gamma, beta):
    idx2d = cate_x.reshape(IDX_ROWS, 128)
    slabs = _sc_gather(idx2d, emb_table)
    x3 = slabs.reshape(NSLAB, HALF, 128)
    out = _tc_proj(x3, W, b, gamma, beta)
    return out.reshape(BATCH, MAX_SEQ_LEN, PROJ_DIM)


# speedup vs baseline: 1.1056x; 1.0042x over previous
import functools

import jax
import jax.numpy as jnp
from jax import lax
from jax.experimental import pallas as pl
from jax.experimental.pallas import tpu as pltpu
from jax.experimental.pallas import tpu_sc as plsc

EMB_DIM = 32
CATE_NUM = 26
PROJ_DIM = 128
MAX_SEQ_LEN = 50
BATCH = 1024

N_ROWS = BATCH * MAX_SEQ_LEN * CATE_NUM
IN_DIM = EMB_DIM * CATE_NUM
BL = BATCH * MAX_SEQ_LEN
HALF = BL // 2
NSLAB = 13
QUADS = 4

NUM_CORES = 2
NUM_SUBCORES = 16
NUM_TILES = NUM_CORES * NUM_SUBCORES
P_PER_TILE = HALF // NUM_TILES
CHUNK = 128
GROUP = 5
CHUNKS_PER_SLAB = P_PER_TILE * QUADS // CHUNK
CHUNKS_PER_TILE = NSLAB * CHUNKS_PER_SLAB
GROUPS_PER_SLAB = CHUNKS_PER_SLAB // GROUP
GROUPS_PER_TILE = NSLAB * GROUPS_PER_SLAB
GROUP_ROWS = GROUP * CHUNK
SLAB_ROWS = HALF * QUADS


IDX_ROWS = BATCH * MAX_SEQ_LEN * CATE_NUM // 128
TILE_WORDS = P_PER_TILE * CATE_NUM
STAGE = 176


def _sc_gather(idx2d, table):
    mesh = plsc.VectorSubcoreMesh(core_axis_name="c", subcore_axis_name="s")

    @functools.partial(
        pl.kernel,
        mesh=mesh,
        compiler_params=pltpu.CompilerParams(use_tc_tiling_on_sc=False,
                                             needs_layout_passes=False),
        out_type=jax.ShapeDtypeStruct((NSLAB, SLAB_ROWS, EMB_DIM),
                                      jnp.float32),
        scratch_types=[
            pltpu.VMEM((2, STAGE, 128), jnp.int32),
            pltpu.VMEM((2, GROUP, CHUNK), jnp.int32),
            pltpu.VMEM((3, GROUP_ROWS, EMB_DIM), jnp.float32),
            pltpu.SemaphoreType.DMA,
            pltpu.SemaphoreType.DMA,
        ],
    )
    def k(table_hbm, idx_hbm, out_hbm, idx_stage, idx_buf, rows_v, gsem, ssem):
        wid = lax.axis_index("s") * NUM_CORES + lax.axis_index("c")
        base_top = wid * TILE_WORDS
        base_bot = HALF * CATE_NUM + wid * TILE_WORDS
        r_top = jnp.minimum((base_top >> 7) & ~7, IDX_ROWS - STAGE)
        r_bot = jnp.minimum((base_bot >> 7) & ~7, IDX_ROWS - STAGE)
        pltpu.sync_copy(idx_hbm.at[pl.ds(r_top, STAGE)], idx_stage.at[0])
        pltpu.sync_copy(idx_hbm.at[pl.ds(r_bot, STAGE)], idx_stage.at[1])
        off_top = base_top - r_top * 128
        off_bot = base_bot - r_bot * 128
        row_base = wid * P_PER_TILE * QUADS
        iota = lax.iota(jnp.int32, 16)
        qv = lax.bitwise_and(iota, 3)
        dv = lax.shift_right_logical(iota, 2)

        def transform(g, pb):
            j = g // GROUPS_PER_SLAB
            s5 = g - j * GROUPS_PER_SLAB
            c2 = 4 * j + qv
            is_bot = c2 >= CATE_NUM
            half = jnp.where(is_bot, 1, 0)
            cadj = jnp.where(is_bot, c2 - CATE_NUM + off_bot, c2 + off_top)
            for i in range(GROUP):
                dp0 = (s5 * GROUP + i) * 32
                for o in range(CHUNK // 16):
                    word = (dp0 + o * 4 + dv) * CATE_NUM + cadj
                    vals = plsc.load_gather(
                        idx_stage,
                        [half, lax.shift_right_logical(word, 7),
                         lax.bitwise_and(word, 127)])
                    idx_buf[pb, i, pl.ds(o * 16, 16)] = vals

        transform(0, 0)

        def body(g, carry):
            j = g // GROUPS_PER_SLAB
            s5 = g - j * GROUPS_PER_SLAB
            p3 = lax.rem(g, 3)
            pi = lax.rem(g, 2)
            buf = rows_v.at[p3]
            @pl.when(g >= 3)
            def _():
                pltpu.make_async_copy(
                    rows_v.at[0],
                    out_hbm.at[0, pl.ds(0, GROUP_ROWS)],
                    ssem).wait()
            cps = []
            for i in range(GROUP):
                cps.append(pltpu.async_copy(
                    table_hbm.at[idx_buf.at[pi, i]],
                    buf.at[pl.ds(i * CHUNK, CHUNK)],
                    gsem))
            transform(g + 1, 1 - pi)
            for cp in cps:
                cp.wait()
            pltpu.async_copy(
                buf,
                out_hbm.at[j, pl.ds(row_base + s5 * GROUP_ROWS, GROUP_ROWS)],
                ssem)
            return carry

        lax.fori_loop(0, GROUPS_PER_TILE, body, 0)
        for _ in range(3):
            pltpu.make_async_copy(
                rows_v.at[0],
                out_hbm.at[0, pl.ds(0, GROUP_ROWS)],
                ssem).wait()

    return k(table, idx2d)


BB = 64
PAIRS = BB * MAX_SEQ_LEN
XW = 7 * 128


def _tc_proj_body(x_ref, wt_ref, wb_ref, b_ref, g_ref, be_ref, o_ref):
    x3 = x_ref[...]
    x_top = jnp.concatenate([x3[j] for j in range(7)], axis=1)
    x_bot = jnp.concatenate([x3[j] for j in range(6, 13)], axis=1)
    bvec = b_ref[...]
    gvec = g_ref[...]
    bevec = be_ref[...]

    def norm(h):
        h = h + bvec
        mu = jnp.mean(h, axis=1, keepdims=True)
        d = h - mu
        var = jnp.mean(d * d, axis=1, keepdims=True)
        out = d * lax.rsqrt(var + 1e-5) * gvec + bevec
        return out.reshape(BB, MAX_SEQ_LEN, PROJ_DIM)

    h_top = jnp.dot(x_top, wt_ref[...], preferred_element_type=jnp.float32)
    h_bot = jnp.dot(x_bot, wb_ref[...], preferred_element_type=jnp.float32)
    o_ref[0] = norm(h_top)
    o_ref[1] = norm(h_bot)


def _tc_proj(x3, W, b, gamma, beta):
    w_top = jnp.concatenate([W, jnp.zeros((64, PROJ_DIM), W.dtype)], axis=0)
    w_bot = jnp.concatenate([jnp.zeros((64, PROJ_DIM), W.dtype), W], axis=0)
    grid = (BATCH // 2 // BB,)
    return pl.pallas_call(
        _tc_proj_body,
        grid=grid,
        in_specs=[
            pl.BlockSpec((NSLAB, PAIRS, 128), lambda i: (0, i, 0)),
            pl.BlockSpec((XW, PROJ_DIM), lambda i: (0, 0)),
            pl.BlockSpec((XW, PROJ_DIM), lambda i: (0, 0)),
            pl.BlockSpec((1, PROJ_DIM), lambda i: (0, 0)),
            pl.BlockSpec((1, PROJ_DIM), lambda i: (0, 0)),
            pl.BlockSpec((1, PROJ_DIM), lambda i: (0, 0)),
        ],
        out_specs=pl.BlockSpec((2, BB, MAX_SEQ_LEN, PROJ_DIM),
                               lambda i: (0, i, 0, 0)),
        out_shape=jax.ShapeDtypeStruct(
            (2, BATCH // 2, MAX_SEQ_LEN, PROJ_DIM), jnp.float32),
    )(x3, w_top, w_bot, b.reshape(1, PROJ_DIM), gamma.reshape(1, PROJ_DIM),
      beta.reshape(1, PROJ_DIM))


def kernel(cate_x, emb_table, W, b, gamma, beta):
    idx2d = cate_x.reshape(IDX_ROWS, 128)
    slabs = _sc_gather(idx2d, emb_table)
    x3 = slabs.reshape(NSLAB, HALF, 128)
    out = _tc_proj(x3, W, b, gamma, beta)
    return out.reshape(BATCH, MAX_SEQ_LEN, PROJ_DIM)


# speedup vs baseline: 1.1899x; 1.0763x over previous
import functools

import jax
import jax.numpy as jnp
from jax import lax
from jax.experimental import pallas as pl
from jax.experimental.pallas import tpu as pltpu
from jax.experimental.pallas import tpu_sc as plsc

EMB_DIM = 32
CATE_NUM = 26
PROJ_DIM = 128
MAX_SEQ_LEN = 50
BATCH = 1024

N_ROWS = BATCH * MAX_SEQ_LEN * CATE_NUM
IN_DIM = EMB_DIM * CATE_NUM
BL = BATCH * MAX_SEQ_LEN
HALF = BL // 2
NSLAB = 13
QUADS = 4

NUM_CORES = 2
NUM_SUBCORES = 16
NUM_TILES = NUM_CORES * NUM_SUBCORES
P_PER_TILE = HALF // NUM_TILES
CHUNK = 128
GROUP = 5
CHUNKS_PER_SLAB = P_PER_TILE * QUADS // CHUNK
CHUNKS_PER_TILE = NSLAB * CHUNKS_PER_SLAB
GROUPS_PER_SLAB = CHUNKS_PER_SLAB // GROUP
GROUPS_PER_TILE = NSLAB * GROUPS_PER_SLAB
GROUP_ROWS = GROUP * CHUNK
SLAB_ROWS = HALF * QUADS


IDX_ROWS = BATCH * MAX_SEQ_LEN * CATE_NUM // 128
TILE_WORDS = P_PER_TILE * CATE_NUM
STAGE = 176


def _sc_gather(idx2d, table):
    mesh = plsc.VectorSubcoreMesh(core_axis_name="c", subcore_axis_name="s")

    @functools.partial(
        pl.kernel,
        mesh=mesh,
        compiler_params=pltpu.CompilerParams(use_tc_tiling_on_sc=False,
                                             needs_layout_passes=False),
        out_type=jax.ShapeDtypeStruct((NSLAB, SLAB_ROWS, EMB_DIM),
                                      jnp.float32),
        scratch_types=[
            pltpu.VMEM((2, STAGE, 128), jnp.int32),
            pltpu.VMEM((3, GROUP, CHUNK), jnp.int32),
            pltpu.VMEM((3, GROUP_ROWS, EMB_DIM), jnp.float32),
            pltpu.SemaphoreType.DMA((2,)),
            pltpu.SemaphoreType.DMA,
        ],
    )
    def k(table_hbm, idx_hbm, out_hbm, idx_stage, idx_buf, rows_v, gsem, ssem):
        wid = lax.axis_index("s") * NUM_CORES + lax.axis_index("c")
        base_top = wid * TILE_WORDS
        base_bot = HALF * CATE_NUM + wid * TILE_WORDS
        r_top = jnp.minimum((base_top >> 7) & ~7, IDX_ROWS - STAGE)
        r_bot = jnp.minimum((base_bot >> 7) & ~7, IDX_ROWS - STAGE)
        pltpu.sync_copy(idx_hbm.at[pl.ds(r_top, STAGE)], idx_stage.at[0])
        pltpu.sync_copy(idx_hbm.at[pl.ds(r_bot, STAGE)], idx_stage.at[1])
        off_top = base_top - r_top * 128
        off_bot = base_bot - r_bot * 128
        row_base = wid * P_PER_TILE * QUADS
        iota = lax.iota(jnp.int32, 16)
        qv = lax.bitwise_and(iota, 3)
        dv = lax.shift_right_logical(iota, 2)

        def transform(g, pb):
            j = g // GROUPS_PER_SLAB
            s5 = g - j * GROUPS_PER_SLAB
            c2 = 4 * j + qv
            is_bot = c2 >= CATE_NUM
            half = jnp.where(is_bot, 1, 0)
            cadj = jnp.where(is_bot, c2 - CATE_NUM + off_bot, c2 + off_top)
            for i in range(GROUP):
                dp0 = (s5 * GROUP + i) * 32
                for o in range(CHUNK // 16):
                    word = (dp0 + o * 4 + dv) * CATE_NUM + cadj
                    vals = plsc.load_gather(
                        idx_stage,
                        [half, lax.shift_right_logical(word, 7),
                         lax.bitwise_and(word, 127)])
                    idx_buf[pb, i, pl.ds(o * 16, 16)] = vals

        def fire(g3, gi):
            for i in range(GROUP):
                pltpu.async_copy(
                    table_hbm.at[idx_buf.at[g3, i]],
                    rows_v.at[g3].at[pl.ds(i * CHUNK, CHUNK)],
                    gsem.at[gi])

        transform(0, 0)
        transform(1, 1)
        fire(0, 0)

        def body(g, carry):
            j = g // GROUPS_PER_SLAB
            s5 = g - j * GROUPS_PER_SLAB
            p3 = lax.rem(g, 3)
            pn3 = lax.rem(g + 1, 3)
            pi = lax.rem(g, 2)
            @pl.when(g >= 2)
            def _():
                pltpu.make_async_copy(
                    rows_v.at[0],
                    out_hbm.at[0, pl.ds(0, GROUP_ROWS)],
                    ssem).wait()
            @pl.when(g + 1 < GROUPS_PER_TILE)
            def _():
                fire(pn3, 1 - pi)
            transform(g + 2, lax.rem(g + 2, 3))
            for _ in range(GROUP):
                pltpu.make_async_copy(
                    table_hbm.at[pl.ds(0, CHUNK)],
                    rows_v.at[0].at[pl.ds(0, CHUNK)],
                    gsem.at[pi]).wait()
            pltpu.async_copy(
                rows_v.at[p3],
                out_hbm.at[j, pl.ds(row_base + s5 * GROUP_ROWS, GROUP_ROWS)],
                ssem)
            return carry

        lax.fori_loop(0, GROUPS_PER_TILE, body, 0)
        for _ in range(2):
            pltpu.make_async_copy(
                rows_v.at[0],
                out_hbm.at[0, pl.ds(0, GROUP_ROWS)],
                ssem).wait()

    return k(table, idx2d)


BB = 64
PAIRS = BB * MAX_SEQ_LEN
XW = 7 * 128


def _tc_proj_body(x_ref, wt_ref, wb_ref, b_ref, g_ref, be_ref, o_ref):
    x3 = x_ref[...]
    x_top = jnp.concatenate([x3[j] for j in range(7)], axis=1)
    x_bot = jnp.concatenate([x3[j] for j in range(6, 13)], axis=1)
    bvec = b_ref[...]
    gvec = g_ref[...]
    bevec = be_ref[...]

    def norm(h):
        h = h + bvec
        mu = jnp.mean(h, axis=1, keepdims=True)
        d = h - mu
        var = jnp.mean(d * d, axis=1, keepdims=True)
        out = d * lax.rsqrt(var + 1e-5) * gvec + bevec
        return out.reshape(BB, MAX_SEQ_LEN, PROJ_DIM)

    h_top = jnp.dot(x_top, wt_ref[...], preferred_element_type=jnp.float32)
    h_bot = jnp.dot(x_bot, wb_ref[...], preferred_element_type=jnp.float32)
    o_ref[0] = norm(h_top)
    o_ref[1] = norm(h_bot)


def _tc_proj(x3, W, b, gamma, beta):
    w_top = jnp.concatenate([W, jnp.zeros((64, PROJ_DIM), W.dtype)], axis=0)
    w_bot = jnp.concatenate([jnp.zeros((64, PROJ_DIM), W.dtype), W], axis=0)
    grid = (BATCH // 2 // BB,)
    return pl.pallas_call(
        _tc_proj_body,
        grid=grid,
        in_specs=[
            pl.BlockSpec((NSLAB, PAIRS, 128), lambda i: (0, i, 0)),
            pl.BlockSpec((XW, PROJ_DIM), lambda i: (0, 0)),
            pl.BlockSpec((XW, PROJ_DIM), lambda i: (0, 0)),
            pl.BlockSpec((1, PROJ_DIM), lambda i: (0, 0)),
            pl.BlockSpec((1, PROJ_DIM), lambda i: (0, 0)),
            pl.BlockSpec((1, PROJ_DIM), lambda i: (0, 0)),
        ],
        out_specs=pl.BlockSpec((2, BB, MAX_SEQ_LEN, PROJ_DIM),
                               lambda i: (0, i, 0, 0)),
        out_shape=jax.ShapeDtypeStruct(
            (2, BATCH // 2, MAX_SEQ_LEN, PROJ_DIM), jnp.float32),
    )(x3, w_top, w_bot, b.reshape(1, PROJ_DIM), gamma.reshape(1, PROJ_DIM),
      beta.reshape(1, PROJ_DIM))


def kernel(cate_x, emb_table, W, b, gamma, beta):
    idx2d = cate_x.reshape(IDX_ROWS, 128)
    slabs = _sc_gather(idx2d, emb_table)
    x3 = slabs.reshape(NSLAB, HALF, 128)
    out = _tc_proj(x3, W, b, gamma, beta)
    return out.reshape(BATCH, MAX_SEQ_LEN, PROJ_DIM)
